# Initial kernel scaffold; baseline (speedup 1.0000x reference)
import jax
import jax.numpy as jnp
from jax.experimental import pallas as pl


def kernel(hidden, sx, aa, bb, pp, time_decay, time_first, time_mix_key, time_mix_value, time_mix_receptance, Wk, Wv, Wr, Wo):
    raise NotImplementedError("write your pallas kernel here")


# speedup vs baseline: 21.2363x; 21.2363x over previous
import jax
import jax.numpy as jnp
from jax.experimental import pallas as pl
from jax.experimental.pallas import tpu as pltpu

T = 4096
H = 2048


_PROJ_TM = 256
_PROJ_NI = (T // _PROJ_TM) // 2


def _proj_body(tm_ref, h_ref, cx_ref, w_ref, o_ref, *, sig):
    tm = tm_ref[...]
    x = h_ref[...] * tm + cx_ref[...] * (1.0 - tm)
    acc = jnp.dot(x, w_ref[...], preferred_element_type=jnp.float32)
    if sig:
        acc = jax.nn.sigmoid(acc)
    o_ref[...] = acc


def _projection(tm, h, cx, w, sig):
    import functools
    body = functools.partial(_proj_body, sig=sig)
    return pl.pallas_call(
        body,
        out_shape=jax.ShapeDtypeStruct((T, H), jnp.float32),
        grid=(2, _PROJ_NI),
        in_specs=[
            pl.BlockSpec((1, H), lambda c, i: (0, 0)),
            pl.BlockSpec((_PROJ_TM, H), lambda c, i: (c * _PROJ_NI + i, 0)),
            pl.BlockSpec((_PROJ_TM, H), lambda c, i: (c * _PROJ_NI + i, 0)),
            pl.BlockSpec((H, H), lambda c, i: (0, 0)),
        ],
        out_specs=pl.BlockSpec((_PROJ_TM, H), lambda c, i: (c * _PROJ_NI + i, 0)),
        compiler_params=pltpu.CompilerParams(
            dimension_semantics=("parallel", "arbitrary"),
            vmem_limit_bytes=58 * 1024 * 1024,
        ),
        name="rwkv_proj",
    )(tm, h, cx, w)


_SC_TC = 512
_SC_NT = T // _SC_TC


def _scan_body(k_ref, v_ref, aa0_ref, bb0_ref, pp0_ref, tf_ref, w_ref,
               wkv_ref, aa_ref, bb_ref, pp_ref):
    t_idx = pl.program_id(1)

    @pl.when(t_idx == 0)
    def _():
        aa_ref[...] = aa0_ref[...]
        bb_ref[...] = bb0_ref[...]
        pp_ref[...] = pp0_ref[...]

    tf = tf_ref[...]
    w = w_ref[...]

    def step(t, carry):
        aa, bb, ipp = carry
        kk = k_ref[t]
        vv = v_ref[t]
        ww = tf + kk
        p = jnp.maximum(ipp, ww)
        e1 = jnp.exp(ipp - p)
        e2 = jnp.exp(ww - p)
        wkv_ref[t] = (e1 * aa + e2 * vv) / (e1 * bb + e2)
        ww2 = w + ipp
        p2 = jnp.maximum(ww2, kk)
        e1b = jnp.exp(ww2 - p2)
        e2b = jnp.exp(kk - p2)
        return (e1b * aa + e2b * vv, e1b * bb + e2b, p2)

    init = (aa_ref[...], bb_ref[...], pp_ref[...])
    aa, bb, pp = jax.lax.fori_loop(0, _SC_TC, step, init)
    aa_ref[...] = aa
    bb_ref[...] = bb
    pp_ref[...] = pp


def _wkv_scan(k3, v3, aa3, bb3, pp3, tf3, w3):
    blk = pl.BlockSpec((_SC_TC, 8, 128), lambda c, t: (t, c, 0))
    st = pl.BlockSpec((8, 128), lambda c, t: (c, 0))
    return pl.pallas_call(
        _scan_body,
        out_shape=(
            jax.ShapeDtypeStruct((T, 16, 128), jnp.float32),
            jax.ShapeDtypeStruct((16, 128), jnp.float32),
            jax.ShapeDtypeStruct((16, 128), jnp.float32),
            jax.ShapeDtypeStruct((16, 128), jnp.float32),
        ),
        grid=(2, _SC_NT),
        in_specs=[blk, blk, st, st, st, st, st],
        out_specs=(blk, st, st, st),
        compiler_params=pltpu.CompilerParams(
            dimension_semantics=("parallel", "arbitrary"),
            vmem_limit_bytes=58 * 1024 * 1024,
        ),
        name="rwkv_wkv_scan",
    )(k3, v3, aa3, bb3, pp3, tf3, w3)


_OUT_TM = 256
_OUT_NI = (T // _OUT_TM) // 2


def _out_body(h_ref, r_ref, wkv_ref, wo_ref, o_ref):
    rw = r_ref[...] * wkv_ref[...]
    o_ref[...] = h_ref[...] + jnp.dot(rw, wo_ref[...],
                                      preferred_element_type=jnp.float32)


def _output(h, r, wkv, wo):
    rows = pl.BlockSpec((_OUT_TM, H), lambda c, i: (c * _OUT_NI + i, 0))
    return pl.pallas_call(
        _out_body,
        out_shape=jax.ShapeDtypeStruct((T, H), jnp.float32),
        grid=(2, _OUT_NI),
        in_specs=[rows, rows, rows, pl.BlockSpec((H, H), lambda c, i: (0, 0))],
        out_specs=rows,
        compiler_params=pltpu.CompilerParams(
            dimension_semantics=("parallel", "arbitrary"),
            vmem_limit_bytes=58 * 1024 * 1024,
        ),
        name="rwkv_out",
    )(h, r, wkv, wo)


def kernel(hidden, sx, aa, bb, pp, time_decay, time_first, time_mix_key,
           time_mix_value, time_mix_receptance, Wk, Wv, Wr, Wo):
    cx = jnp.concatenate((sx[None, :], hidden[:-1, :]), axis=0)

    k = _projection(time_mix_key.reshape(1, H), hidden, cx, Wk, sig=False)
    v = _projection(time_mix_value.reshape(1, H), hidden, cx, Wv, sig=False)
    r = _projection(time_mix_receptance.reshape(1, H), hidden, cx, Wr, sig=True)

    w_decay = -jnp.exp(time_decay)
    wkv3, aa3, bb3, pp3 = _wkv_scan(
        k.reshape(T, 16, 128), v.reshape(T, 16, 128),
        aa.reshape(16, 128), bb.reshape(16, 128), pp.reshape(16, 128),
        time_first.reshape(16, 128), w_decay.reshape(16, 128))
    wkv = wkv3.reshape(T, H)

    out = _output(hidden, r, wkv, Wo)
    return out, hidden[-1, :], aa3.reshape(H), bb3.reshape(H), pp3.reshape(H)


# speedup vs baseline: 26.6865x; 1.2566x over previous
import functools
import math

import jax
import jax.numpy as jnp
from jax.experimental import pallas as pl
from jax.experimental.pallas import tpu as pltpu

T = 4096
H = 2048

_LOG2E = math.log2(math.e)


_PROJ_TM = 128
_PROJ_NI = T // _PROJ_TM


def _proj_body(tmk_ref, tmv_ref, tmr_ref, h_ref, cx_ref,
               wk_ref, wv_ref, wr_ref, k_ref, v_ref, r_ref):
    h = h_ref[...]
    cx = cx_ref[...]
    tmk = tmk_ref[...]
    tmv = tmv_ref[...]
    tmr = tmr_ref[...]
    xk = h * tmk + cx * (1.0 - tmk)
    xv = h * tmv + cx * (1.0 - tmv)
    xr = h * tmr + cx * (1.0 - tmr)
    k_ref[...] = jnp.dot(xk, wk_ref[...], preferred_element_type=jnp.float32)
    v_ref[...] = jnp.dot(xv, wv_ref[...], preferred_element_type=jnp.float32)
    r_ref[...] = jax.nn.sigmoid(
        jnp.dot(xr, wr_ref[...], preferred_element_type=jnp.float32))


def _projection(tmk, tmv, tmr, h, cx, wk, wv, wr):
    rows = pl.BlockSpec((_PROJ_TM, H), lambda c, i: (i, 0))
    wblk = pl.BlockSpec((H, H // 2), lambda c, i: (0, c))
    oblk = pl.BlockSpec((_PROJ_TM, H // 2), lambda c, i: (i, c))
    vec = pl.BlockSpec((1, H), lambda c, i: (0, 0))
    out_sds = jax.ShapeDtypeStruct((T, H), jnp.float32)
    return pl.pallas_call(
        _proj_body,
        out_shape=(out_sds, out_sds, out_sds),
        grid=(2, _PROJ_NI),
        in_specs=[vec, vec, vec, rows, rows, wblk, wblk, wblk],
        out_specs=(oblk, oblk, oblk),
        compiler_params=pltpu.CompilerParams(
            dimension_semantics=("parallel", "arbitrary"),
            vmem_limit_bytes=58 * 1024 * 1024,
        ),
        name="rwkv_proj",
    )(tmk, tmv, tmr, h, cx, wk, wv, wr)


_SC_TC = 512
_SC_NT = T // _SC_TC


def _scan_body(k_ref, v_ref, aa0_ref, bb0_ref, pp0_ref, tf_ref, w_ref,
               wkv_ref, aa_ref, bb_ref, pp_ref):
    t_idx = pl.program_id(1)

    @pl.when(t_idx == 0)
    def _():
        aa_ref[...] = aa0_ref[...]
        bb_ref[...] = bb0_ref[...]
        pp_ref[...] = pp0_ref[...]

    tf = tf_ref[...]
    w = w_ref[...]

    def step(t, carry):
        aa, bb, ipp = carry
        kk = k_ref[t]
        vv = v_ref[t]
        ww = tf + kk
        p = jnp.maximum(ipp, ww)
        e1 = jnp.exp2(ipp - p)
        e2 = jnp.exp2(ww - p)
        wkv_ref[t] = (e1 * aa + e2 * vv) / (e1 * bb + e2)
        ww2 = w + ipp
        p2 = jnp.maximum(ww2, kk)
        e1b = jnp.exp2(ww2 - p2)
        e2b = jnp.exp2(kk - p2)
        return (e1b * aa + e2b * vv, e1b * bb + e2b, p2)

    init = (aa_ref[...], bb_ref[...], pp_ref[...])
    aa, bb, pp = jax.lax.fori_loop(0, _SC_TC, step, init, unroll=8)
    aa_ref[...] = aa
    bb_ref[...] = bb
    pp_ref[...] = pp


def _wkv_scan(k3, v3, aa3, bb3, pp3, tf3, w3):
    blk = pl.BlockSpec((_SC_TC, 8, 128), lambda c, t: (t, c, 0))
    st = pl.BlockSpec((8, 128), lambda c, t: (c, 0))
    return pl.pallas_call(
        _scan_body,
        out_shape=(
            jax.ShapeDtypeStruct((T, 16, 128), jnp.float32),
            jax.ShapeDtypeStruct((16, 128), jnp.float32),
            jax.ShapeDtypeStruct((16, 128), jnp.float32),
            jax.ShapeDtypeStruct((16, 128), jnp.float32),
        ),
        grid=(2, _SC_NT),
        in_specs=[blk, blk, st, st, st, st, st],
        out_specs=(blk, st, st, st),
        compiler_params=pltpu.CompilerParams(
            dimension_semantics=("parallel", "arbitrary"),
            vmem_limit_bytes=58 * 1024 * 1024,
        ),
        name="rwkv_wkv_scan",
    )(k3, v3, aa3, bb3, pp3, tf3, w3)


_OUT_TM = 256
_OUT_NI = (T // _OUT_TM) // 2


def _out_body(h_ref, r_ref, wkv_ref, wo_ref, o_ref):
    rw = r_ref[...] * wkv_ref[...]
    o_ref[...] = h_ref[...] + jnp.dot(rw, wo_ref[...],
                                      preferred_element_type=jnp.float32)


def _output(h, r, wkv, wo):
    rows = pl.BlockSpec((_OUT_TM, H), lambda c, i: (c * _OUT_NI + i, 0))
    return pl.pallas_call(
        _out_body,
        out_shape=jax.ShapeDtypeStruct((T, H), jnp.float32),
        grid=(2, _OUT_NI),
        in_specs=[rows, rows, rows, pl.BlockSpec((H, H), lambda c, i: (0, 0))],
        out_specs=rows,
        compiler_params=pltpu.CompilerParams(
            dimension_semantics=("parallel", "arbitrary"),
            vmem_limit_bytes=58 * 1024 * 1024,
        ),
        name="rwkv_out",
    )(h, r, wkv, wo)


def kernel(hidden, sx, aa, bb, pp, time_decay, time_first, time_mix_key,
           time_mix_value, time_mix_receptance, Wk, Wv, Wr, Wo):
    cx = jnp.concatenate((sx[None, :], hidden[:-1, :]), axis=0)

    k, v, r = _projection(
        time_mix_key.reshape(1, H), time_mix_value.reshape(1, H),
        time_mix_receptance.reshape(1, H), hidden, cx,
        Wk * jnp.float32(_LOG2E), Wv, Wr)

    w_decay2 = -jnp.exp(time_decay) * _LOG2E
    tf2 = time_first * _LOG2E
    pp2 = pp * _LOG2E
    wkv3, aa3, bb3, pp3 = _wkv_scan(
        k.reshape(T, 16, 128), v.reshape(T, 16, 128),
        aa.reshape(16, 128), bb.reshape(16, 128), pp2.reshape(16, 128),
        tf2.reshape(16, 128), w_decay2.reshape(16, 128))
    wkv = wkv3.reshape(T, H)

    out = _output(hidden, r, wkv, Wo)
    return (out, hidden[-1, :], aa3.reshape(H), bb3.reshape(H),
            pp3.reshape(H) * jnp.float32(1.0 / _LOG2E))


# speedup vs baseline: 29.5812x; 1.1085x over previous
import math

import jax
import jax.numpy as jnp
from jax.experimental import pallas as pl
from jax.experimental.pallas import tpu as pltpu

T = 4096
H = 2048

_LOG2E = math.log2(math.e)


_PROJ_TM = 256
_PROJ_NI = T // _PROJ_TM


def _proj_body(tma_ref, tmb_ref, h_ref, cx_ref, wk_hbm, wv_hbm, wr_hbm,
               k_ref, v_ref, r_ref, wk_s, wv_s, wr_s, sems):
    c = pl.program_id(0)
    i = pl.program_id(1)

    @pl.when(i == 0)
    def _():
        cols = pl.ds(c * (H // 2), H // 2)
        for n, (hbm, scr) in enumerate(((wk_hbm, wk_s), (wv_hbm, wv_s),
                                        (wr_hbm, wr_s))):
            pltpu.make_async_copy(hbm.at[:, cols], scr, sems.at[n]).start()
        for n, (hbm, scr) in enumerate(((wk_hbm, wk_s), (wv_hbm, wv_s),
                                        (wr_hbm, wr_s))):
            pltpu.make_async_copy(hbm.at[:, cols], scr, sems.at[n]).wait()

    h = h_ref[...]
    cx = cx_ref[...]
    xk = h * tma_ref[0:1] + cx * tmb_ref[0:1]
    xv = h * tma_ref[1:2] + cx * tmb_ref[1:2]
    xr = h * tma_ref[2:3] + cx * tmb_ref[2:3]
    k_ref[...] = jnp.dot(xk, wk_s[...], preferred_element_type=jnp.float32)
    v_ref[...] = jnp.dot(xv, wv_s[...], preferred_element_type=jnp.float32)
    r_ref[...] = jax.nn.sigmoid(
        jnp.dot(xr, wr_s[...], preferred_element_type=jnp.float32)
    ).astype(jnp.bfloat16)


def _projection(tma, tmb, h, cx, wk, wv, wr):
    rows = pl.BlockSpec((_PROJ_TM, H), lambda c, i: (i, 0))
    wany = pl.BlockSpec(memory_space=pl.ANY)
    oblk = pl.BlockSpec((_PROJ_TM, H // 2), lambda c, i: (i, c))
    vec = pl.BlockSpec((3, H), lambda c, i: (0, 0))
    return pl.pallas_call(
        _proj_body,
        out_shape=(
            jax.ShapeDtypeStruct((T, H), jnp.float32),
            jax.ShapeDtypeStruct((T, H), jnp.float32),
            jax.ShapeDtypeStruct((T, H), jnp.bfloat16),
        ),
        grid=(2, _PROJ_NI),
        in_specs=[vec, vec, rows, rows, wany, wany, wany],
        out_specs=(oblk, oblk, oblk),
        scratch_shapes=[
            pltpu.VMEM((H, H // 2), jnp.float32),
            pltpu.VMEM((H, H // 2), jnp.float32),
            pltpu.VMEM((H, H // 2), jnp.float32),
            pltpu.SemaphoreType.DMA((3,)),
        ],
        compiler_params=pltpu.CompilerParams(
            dimension_semantics=("parallel", "arbitrary"),
            vmem_limit_bytes=58 * 1024 * 1024,
        ),
        name="rwkv_proj",
    )(tma, tmb, h, cx, wk, wv, wr)


_SC_TC = 512
_SC_NT = T // _SC_TC


def _scan_body(k_ref, v_ref, aa0_ref, bb0_ref, pp0_ref, tf_ref, w_ref,
               wkv_ref, aa_ref, bb_ref, pp_ref):
    t_idx = pl.program_id(1)

    @pl.when(t_idx == 0)
    def _():
        aa_ref[...] = aa0_ref[...]
        bb_ref[...] = bb0_ref[...]
        pp_ref[...] = pp0_ref[...]

    tf = tf_ref[...]
    w = w_ref[...]

    def step(t, carry):
        aa, bb, ipp = carry
        kk = k_ref[t]
        vv = v_ref[t]
        ww = tf + kk
        p = jnp.maximum(ipp, ww)
        e1 = jnp.exp2(ipp - p)
        e2 = jnp.exp2(ww - p)
        wkv_ref[t] = ((e1 * aa + e2 * vv) / (e1 * bb + e2)).astype(jnp.bfloat16)
        ww2 = w + ipp
        p2 = jnp.maximum(ww2, kk)
        e1b = jnp.exp2(ww2 - p2)
        e2b = jnp.exp2(kk - p2)
        return (e1b * aa + e2b * vv, e1b * bb + e2b, p2)

    init = (aa_ref[...], bb_ref[...], pp_ref[...])
    aa, bb, pp = jax.lax.fori_loop(0, _SC_TC, step, init, unroll=8)
    aa_ref[...] = aa
    bb_ref[...] = bb
    pp_ref[...] = pp


def _wkv_scan(k3, v3, aa3, bb3, pp3, tf3, w3):
    blk = pl.BlockSpec((_SC_TC, 8, 128), lambda c, t: (t, c, 0))
    st = pl.BlockSpec((8, 128), lambda c, t: (c, 0))
    return pl.pallas_call(
        _scan_body,
        out_shape=(
            jax.ShapeDtypeStruct((T, 16, 128), jnp.bfloat16),
            jax.ShapeDtypeStruct((16, 128), jnp.float32),
            jax.ShapeDtypeStruct((16, 128), jnp.float32),
            jax.ShapeDtypeStruct((16, 128), jnp.float32),
        ),
        grid=(2, _SC_NT),
        in_specs=[blk, blk, st, st, st, st, st],
        out_specs=(blk, st, st, st),
        compiler_params=pltpu.CompilerParams(
            dimension_semantics=("parallel", "arbitrary"),
            vmem_limit_bytes=58 * 1024 * 1024,
        ),
        name="rwkv_wkv_scan",
    )(k3, v3, aa3, bb3, pp3, tf3, w3)


_OUT_TM = 256
_OUT_NI = (T // _OUT_TM) // 2


def _out_body(h_ref, r_ref, wkv_ref, wo_ref, o_ref):
    rw = r_ref[...] * wkv_ref[...]
    o_ref[...] = h_ref[...] + jnp.dot(rw, wo_ref[...],
                                      preferred_element_type=jnp.float32)


def _output(h, r, wkv, wo):
    def rows(dt):
        return pl.BlockSpec((_OUT_TM, H), lambda c, i: (c * _OUT_NI + i, 0))
    return pl.pallas_call(
        _out_body,
        out_shape=jax.ShapeDtypeStruct((T, H), jnp.float32),
        grid=(2, _OUT_NI),
        in_specs=[rows(jnp.float32), rows(jnp.bfloat16), rows(jnp.bfloat16),
                  pl.BlockSpec((H, H), lambda c, i: (0, 0))],
        out_specs=rows(jnp.float32),
        compiler_params=pltpu.CompilerParams(
            dimension_semantics=("parallel", "arbitrary"),
            vmem_limit_bytes=58 * 1024 * 1024,
        ),
        name="rwkv_out",
    )(h, r, wkv, wo)


def kernel(hidden, sx, aa, bb, pp, time_decay, time_first, time_mix_key,
           time_mix_value, time_mix_receptance, Wk, Wv, Wr, Wo):
    cx = jnp.concatenate((sx[None, :], hidden[:-1, :]), axis=0)

    s = jnp.float32(_LOG2E)
    tma = jnp.stack((time_mix_key * s, time_mix_value,
                     time_mix_receptance))
    tmb = jnp.stack(((1.0 - time_mix_key) * s, 1.0 - time_mix_value,
                     1.0 - time_mix_receptance))
    k, v, r = _projection(tma, tmb, hidden, cx, Wk, Wv, Wr)

    w_decay2 = -jnp.exp(time_decay) * _LOG2E
    tf2 = time_first * _LOG2E
    pp2 = pp * _LOG2E
    wkv3, aa3, bb3, pp3 = _wkv_scan(
        k.reshape(T, 16, 128), v.reshape(T, 16, 128),
        aa.reshape(16, 128), bb.reshape(16, 128), pp2.reshape(16, 128),
        tf2.reshape(16, 128), w_decay2.reshape(16, 128))
    wkv = wkv3.reshape(T, H)

    out = _output(hidden, r, wkv, Wo)
    return (out, hidden[-1, :], aa3.reshape(H), bb3.reshape(H),
            pp3.reshape(H) * jnp.float32(1.0 / _LOG2E))


# speedup vs baseline: 39.9317x; 1.3499x over previous
import math

import jax
import jax.numpy as jnp
from jax.experimental import pallas as pl
from jax.experimental.pallas import tpu as pltpu

T = 4096
H = 2048

_LOG2E = math.log2(math.e)


_PROJ_TM = 256
_PROJ_NI = T // _PROJ_TM


def _proj_body(tma_ref, tmb_ref, sx_ref, h_ref, hp_ref,
               wk_hbm, wv_hbm, wr_hbm,
               k_ref, v_ref, r_ref, wk_s, wv_s, wr_s, sems):
    c = pl.program_id(0)
    i = pl.program_id(1)

    @pl.when(i == 0)
    def _():
        cols = pl.ds(c * (H // 2), H // 2)
        for n, (hbm, scr) in enumerate(((wk_hbm, wk_s), (wv_hbm, wv_s),
                                        (wr_hbm, wr_s))):
            pltpu.make_async_copy(hbm.at[:, cols], scr, sems.at[n]).start()
        for n, (hbm, scr) in enumerate(((wk_hbm, wk_s), (wv_hbm, wv_s),
                                        (wr_hbm, wr_s))):
            pltpu.make_async_copy(hbm.at[:, cols], scr, sems.at[n]).wait()

    h = h_ref[...]
    first = jnp.where(i == 0, sx_ref[...], hp_ref[_PROJ_TM - 1:_PROJ_TM, :])
    cx = jnp.concatenate((first, h[:-1, :]), axis=0)
    xk = h * tma_ref[0:1] + cx * tmb_ref[0:1]
    xv = h * tma_ref[1:2] + cx * tmb_ref[1:2]
    xr = h * tma_ref[2:3] + cx * tmb_ref[2:3]
    k_ref[...] = jnp.dot(xk, wk_s[...], preferred_element_type=jnp.float32)
    v_ref[...] = jnp.dot(xv, wv_s[...], preferred_element_type=jnp.float32)
    r_ref[...] = jax.nn.sigmoid(
        jnp.dot(xr, wr_s[...], preferred_element_type=jnp.float32)
    ).astype(jnp.bfloat16)


def _projection(tma, tmb, sx2, h, wk, wv, wr):
    rows = pl.BlockSpec((_PROJ_TM, H), lambda c, i: (i, 0))
    rows_prev = pl.BlockSpec((_PROJ_TM, H),
                             lambda c, i: (jnp.maximum(i - 1, 0), 0))
    wany = pl.BlockSpec(memory_space=pl.ANY)
    oblk = pl.BlockSpec((_PROJ_TM, H // 2), lambda c, i: (i, c))
    vec = pl.BlockSpec((3, H), lambda c, i: (0, 0))
    svec = pl.BlockSpec((1, H), lambda c, i: (0, 0))
    return pl.pallas_call(
        _proj_body,
        out_shape=(
            jax.ShapeDtypeStruct((T, H), jnp.float32),
            jax.ShapeDtypeStruct((T, H), jnp.float32),
            jax.ShapeDtypeStruct((T, H), jnp.bfloat16),
        ),
        grid=(2, _PROJ_NI),
        in_specs=[vec, vec, svec, rows, rows_prev, wany, wany, wany],
        out_specs=(oblk, oblk, oblk),
        scratch_shapes=[
            pltpu.VMEM((H, H // 2), jnp.float32),
            pltpu.VMEM((H, H // 2), jnp.float32),
            pltpu.VMEM((H, H // 2), jnp.float32),
            pltpu.SemaphoreType.DMA((3,)),
        ],
        compiler_params=pltpu.CompilerParams(
            dimension_semantics=("arbitrary", "arbitrary"),
            vmem_limit_bytes=58 * 1024 * 1024,
        ),
        name="rwkv_proj",
    )(tma, tmb, sx2, h, h, wk, wv, wr)


_SC_TC = 512
_SC_NT = T // _SC_TC


def _in_copies(src_hbm, buf, sems, slot, chunk, c):
    t0 = chunk * _SC_TC
    for g in range(8):
        col = (c * 8 + g) * 128
        yield pltpu.make_async_copy(
            src_hbm.at[pl.ds(t0, _SC_TC), pl.ds(col, 128)],
            buf.at[slot, :, g, :],
            sems.at[slot, g])


def _out_copies(dst_hbm, buf, sems, slot, chunk, c):
    t0 = chunk * _SC_TC
    for g in range(8):
        col = (c * 8 + g) * 128
        yield pltpu.make_async_copy(
            buf.at[slot, :, g, :],
            dst_hbm.at[pl.ds(t0, _SC_TC), pl.ds(col, 128)],
            sems.at[slot, g])


def _scan_body(k_hbm, v_hbm, aa0_ref, bb0_ref, pp0_ref, tf_ref, w_ref,
               wkv_hbm, aa_ref, bb_ref, pp_ref,
               kbuf, vbuf, obuf, ksem, vsem, osem):
    c = pl.program_id(0)
    t = pl.program_id(1)
    slot = jax.lax.rem(t, 2)
    nslot = jax.lax.rem(t + 1, 2)

    @pl.when(t == 0)
    def _():
        for cp in _in_copies(k_hbm, kbuf, ksem, 0, 0, c):
            cp.start()
        for cp in _in_copies(v_hbm, vbuf, vsem, 0, 0, c):
            cp.start()
        aa_ref[...] = aa0_ref[...]
        bb_ref[...] = bb0_ref[...]
        pp_ref[...] = pp0_ref[...]

    @pl.when(t + 1 < _SC_NT)
    def _():
        for cp in _in_copies(k_hbm, kbuf, ksem, nslot, t + 1, c):
            cp.start()
        for cp in _in_copies(v_hbm, vbuf, vsem, nslot, t + 1, c):
            cp.start()

    @pl.when(t >= 2)
    def _():
        for cp in _out_copies(wkv_hbm, obuf, osem, slot, t - 2, c):
            cp.wait()

    for cp in _in_copies(k_hbm, kbuf, ksem, slot, t, c):
        cp.wait()
    for cp in _in_copies(v_hbm, vbuf, vsem, slot, t, c):
        cp.wait()

    tf = tf_ref[...]
    w = w_ref[...]

    def step(tt, carry):
        aa, bb, ipp = carry
        kk = kbuf[slot, tt]
        vv = vbuf[slot, tt]
        ww = tf + kk
        p = jnp.maximum(ipp, ww)
        e1 = jnp.exp2(ipp - p)
        e2 = jnp.exp2(ww - p)
        obuf[slot, tt] = (e1 * aa + e2 * vv) / (e1 * bb + e2)
        ww2 = w + ipp
        p2 = jnp.maximum(ww2, kk)
        e1b = jnp.exp2(ww2 - p2)
        e2b = jnp.exp2(kk - p2)
        return (e1b * aa + e2b * vv, e1b * bb + e2b, p2)

    init = (aa_ref[...], bb_ref[...], pp_ref[...])
    aa, bb, pp = jax.lax.fori_loop(0, _SC_TC, step, init, unroll=8)
    aa_ref[...] = aa
    bb_ref[...] = bb
    pp_ref[...] = pp

    for cp in _out_copies(wkv_hbm, obuf, osem, slot, t, c):
        cp.start()

    @pl.when(t == _SC_NT - 1)
    def _():
        for cp in _out_copies(wkv_hbm, obuf, osem, nslot, t - 1, c):
            cp.wait()
        for cp in _out_copies(wkv_hbm, obuf, osem, slot, t, c):
            cp.wait()


def _wkv_scan(k2, v2, aa3, bb3, pp3, tf3, w3):
    st = pl.BlockSpec((8, 128), lambda c, t: (c, 0))
    hbm = pl.BlockSpec(memory_space=pl.ANY)
    return pl.pallas_call(
        _scan_body,
        out_shape=(
            jax.ShapeDtypeStruct((T, H), jnp.float32),
            jax.ShapeDtypeStruct((16, 128), jnp.float32),
            jax.ShapeDtypeStruct((16, 128), jnp.float32),
            jax.ShapeDtypeStruct((16, 128), jnp.float32),
        ),
        grid=(2, _SC_NT),
        in_specs=[hbm, hbm, st, st, st, st, st],
        out_specs=(hbm, st, st, st),
        scratch_shapes=[
            pltpu.VMEM((2, _SC_TC, 8, 128), jnp.float32),
            pltpu.VMEM((2, _SC_TC, 8, 128), jnp.float32),
            pltpu.VMEM((2, _SC_TC, 8, 128), jnp.float32),
            pltpu.SemaphoreType.DMA((2, 8)),
            pltpu.SemaphoreType.DMA((2, 8)),
            pltpu.SemaphoreType.DMA((2, 8)),
        ],
        compiler_params=pltpu.CompilerParams(
            dimension_semantics=("arbitrary", "arbitrary"),
            vmem_limit_bytes=58 * 1024 * 1024,
        ),
        name="rwkv_wkv_scan",
    )(k2, v2, aa3, bb3, pp3, tf3, w3)


_OUT_TM = 256
_OUT_NI = (T // _OUT_TM) // 2


def _out_body(h_ref, r_ref, wkv_ref, wo_ref, o_ref):
    rw = r_ref[...] * wkv_ref[...]
    o_ref[...] = h_ref[...] + jnp.dot(rw, wo_ref[...],
                                      preferred_element_type=jnp.float32)


def _output(h, r, wkv, wo):
    rows = pl.BlockSpec((_OUT_TM, H), lambda c, i: (c * _OUT_NI + i, 0))
    return pl.pallas_call(
        _out_body,
        out_shape=jax.ShapeDtypeStruct((T, H), jnp.float32),
        grid=(2, _OUT_NI),
        in_specs=[rows, rows, rows,
                  pl.BlockSpec((H, H), lambda c, i: (0, 0))],
        out_specs=rows,
        compiler_params=pltpu.CompilerParams(
            dimension_semantics=("arbitrary", "arbitrary"),
            vmem_limit_bytes=58 * 1024 * 1024,
        ),
        name="rwkv_out",
    )(h, r, wkv, wo)


def kernel(hidden, sx, aa, bb, pp, time_decay, time_first, time_mix_key,
           time_mix_value, time_mix_receptance, Wk, Wv, Wr, Wo):
    s = jnp.float32(_LOG2E)
    tma = jnp.stack((time_mix_key * s, time_mix_value,
                     time_mix_receptance))
    tmb = jnp.stack(((1.0 - time_mix_key) * s, 1.0 - time_mix_value,
                     1.0 - time_mix_receptance))
    k, v, r = _projection(tma, tmb, sx[None, :], hidden, Wk, Wv, Wr)

    w_decay2 = -jnp.exp(time_decay) * _LOG2E
    tf2 = time_first * _LOG2E
    pp2 = pp * _LOG2E
    wkv, aa3, bb3, pp3 = _wkv_scan(
        k, v, aa.reshape(16, 128), bb.reshape(16, 128), pp2.reshape(16, 128),
        tf2.reshape(16, 128), w_decay2.reshape(16, 128))

    out = _output(hidden, r, wkv, Wo)
    return (out, hidden[-1, :], aa3.reshape(H), bb3.reshape(H),
            pp3.reshape(H) * jnp.float32(1.0 / _LOG2E))


# speedup vs baseline: 40.2963x; 1.0091x over previous
import math

import jax
import jax.numpy as jnp
from jax.experimental import pallas as pl
from jax.experimental.pallas import tpu as pltpu

T = 4096
H = 2048

_LOG2E = math.log2(math.e)


_PROJ_TM = 256
_PROJ_NI = T // _PROJ_TM


def _proj_body(tma_ref, tmb_ref, sx_ref, h_ref,
               wk_hbm, wv_hbm, wr_hbm,
               k_ref, v_ref, r_ref, wk_s, wv_s, wr_s, last_ref, sems):
    c = pl.program_id(0)
    i = pl.program_id(1)

    @pl.when(i == 0)
    def _():
        cols = pl.ds(c * (H // 2), H // 2)
        for n, (hbm, scr) in enumerate(((wk_hbm, wk_s), (wv_hbm, wv_s),
                                        (wr_hbm, wr_s))):
            pltpu.make_async_copy(hbm.at[:, cols], scr, sems.at[n]).start()
        for n, (hbm, scr) in enumerate(((wk_hbm, wk_s), (wv_hbm, wv_s),
                                        (wr_hbm, wr_s))):
            pltpu.make_async_copy(hbm.at[:, cols], scr, sems.at[n]).wait()

    h = h_ref[...]
    first = jnp.where(i == 0, sx_ref[...], last_ref[...])
    last_ref[...] = h[_PROJ_TM - 1:_PROJ_TM, :]
    cx = jnp.concatenate((first, h[:-1, :]), axis=0)
    xk = h * tma_ref[0:1] + cx * tmb_ref[0:1]
    xv = h * tma_ref[1:2] + cx * tmb_ref[1:2]
    xr = h * tma_ref[2:3] + cx * tmb_ref[2:3]
    k_ref[...] = jnp.dot(xk, wk_s[...], preferred_element_type=jnp.float32)
    v_ref[...] = jnp.dot(xv, wv_s[...], preferred_element_type=jnp.float32)
    r_ref[...] = jax.nn.sigmoid(
        jnp.dot(xr, wr_s[...], preferred_element_type=jnp.float32)
    ).astype(jnp.bfloat16)


def _projection(tma, tmb, sx2, h, wk, wv, wr):
    rows = pl.BlockSpec((_PROJ_TM, H), lambda c, i: (i, 0))
    wany = pl.BlockSpec(memory_space=pl.ANY)
    oblk = pl.BlockSpec((_PROJ_TM, H // 2), lambda c, i: (i, c))
    vec = pl.BlockSpec((3, H), lambda c, i: (0, 0))
    svec = pl.BlockSpec((1, H), lambda c, i: (0, 0))
    return pl.pallas_call(
        _proj_body,
        out_shape=(
            jax.ShapeDtypeStruct((T, H), jnp.float32),
            jax.ShapeDtypeStruct((T, H), jnp.float32),
            jax.ShapeDtypeStruct((T, H), jnp.bfloat16),
        ),
        grid=(2, _PROJ_NI),
        in_specs=[vec, vec, svec, rows, wany, wany, wany],
        out_specs=(oblk, oblk, oblk),
        scratch_shapes=[
            pltpu.VMEM((H, H // 2), jnp.float32),
            pltpu.VMEM((H, H // 2), jnp.float32),
            pltpu.VMEM((H, H // 2), jnp.float32),
            pltpu.VMEM((1, H), jnp.float32),
            pltpu.SemaphoreType.DMA((3,)),
        ],
        compiler_params=pltpu.CompilerParams(
            dimension_semantics=("arbitrary", "arbitrary"),
            vmem_limit_bytes=58 * 1024 * 1024,
        ),
        name="rwkv_proj",
    )(tma, tmb, sx2, h, wk, wv, wr)


_SC_TC = 512
_SC_NT = T // _SC_TC


def _in_copies(src_hbm, buf, sems, slot, chunk, c):
    t0 = chunk * _SC_TC
    for g in range(8):
        col = (c * 8 + g) * 128
        yield pltpu.make_async_copy(
            src_hbm.at[pl.ds(t0, _SC_TC), pl.ds(col, 128)],
            buf.at[slot, :, g, :],
            sems.at[slot, g])


def _out_copies(dst_hbm, buf, sems, slot, chunk, c):
    t0 = chunk * _SC_TC
    for g in range(8):
        col = (c * 8 + g) * 128
        yield pltpu.make_async_copy(
            buf.at[slot, :, g, :],
            dst_hbm.at[pl.ds(t0, _SC_TC), pl.ds(col, 128)],
            sems.at[slot, g])


def _scan_body(k_hbm, v_hbm, aa0_ref, bb0_ref, pp0_ref, tf_ref, w_ref,
               wkv_hbm, aa_ref, bb_ref, pp_ref,
               kbuf, vbuf, obuf, ksem, vsem, osem):
    c = pl.program_id(0)
    t = pl.program_id(1)
    slot = jax.lax.rem(t, 2)
    nslot = jax.lax.rem(t + 1, 2)

    @pl.when(t == 0)
    def _():
        for cp in _in_copies(k_hbm, kbuf, ksem, 0, 0, c):
            cp.start()
        for cp in _in_copies(v_hbm, vbuf, vsem, 0, 0, c):
            cp.start()
        aa_ref[...] = aa0_ref[...]
        bb_ref[...] = bb0_ref[...]
        pp_ref[...] = pp0_ref[...]

    @pl.when(t + 1 < _SC_NT)
    def _():
        for cp in _in_copies(k_hbm, kbuf, ksem, nslot, t + 1, c):
            cp.start()
        for cp in _in_copies(v_hbm, vbuf, vsem, nslot, t + 1, c):
            cp.start()

    @pl.when(t >= 2)
    def _():
        for cp in _out_copies(wkv_hbm, obuf, osem, slot, t - 2, c):
            cp.wait()

    for cp in _in_copies(k_hbm, kbuf, ksem, slot, t, c):
        cp.wait()
    for cp in _in_copies(v_hbm, vbuf, vsem, slot, t, c):
        cp.wait()

    tf = tf_ref[...]
    w = w_ref[...]

    def step(tt, carry):
        aa, bb, ipp = carry
        kk = kbuf[slot, tt]
        vv = vbuf[slot, tt]
        ww = tf + kk
        p = jnp.maximum(ipp, ww)
        e1 = jnp.exp2(ipp - p)
        e2 = jnp.exp2(ww - p)
        obuf[slot, tt] = (e1 * aa + e2 * vv) / (e1 * bb + e2)
        ww2 = w + ipp
        p2 = jnp.maximum(ww2, kk)
        e1b = jnp.exp2(ww2 - p2)
        e2b = jnp.exp2(kk - p2)
        return (e1b * aa + e2b * vv, e1b * bb + e2b, p2)

    init = (aa_ref[...], bb_ref[...], pp_ref[...])
    aa, bb, pp = jax.lax.fori_loop(0, _SC_TC, step, init, unroll=8)
    aa_ref[...] = aa
    bb_ref[...] = bb
    pp_ref[...] = pp

    for cp in _out_copies(wkv_hbm, obuf, osem, slot, t, c):
        cp.start()

    @pl.when(t == _SC_NT - 1)
    def _():
        for cp in _out_copies(wkv_hbm, obuf, osem, nslot, t - 1, c):
            cp.wait()
        for cp in _out_copies(wkv_hbm, obuf, osem, slot, t, c):
            cp.wait()


def _wkv_scan(k2, v2, aa3, bb3, pp3, tf3, w3):
    st = pl.BlockSpec((8, 128), lambda c, t: (c, 0))
    hbm = pl.BlockSpec(memory_space=pl.ANY)
    return pl.pallas_call(
        _scan_body,
        out_shape=(
            jax.ShapeDtypeStruct((T, H), jnp.float32),
            jax.ShapeDtypeStruct((16, 128), jnp.float32),
            jax.ShapeDtypeStruct((16, 128), jnp.float32),
            jax.ShapeDtypeStruct((16, 128), jnp.float32),
        ),
        grid=(2, _SC_NT),
        in_specs=[hbm, hbm, st, st, st, st, st],
        out_specs=(hbm, st, st, st),
        scratch_shapes=[
            pltpu.VMEM((2, _SC_TC, 8, 128), jnp.float32),
            pltpu.VMEM((2, _SC_TC, 8, 128), jnp.float32),
            pltpu.VMEM((2, _SC_TC, 8, 128), jnp.float32),
            pltpu.SemaphoreType.DMA((2, 8)),
            pltpu.SemaphoreType.DMA((2, 8)),
            pltpu.SemaphoreType.DMA((2, 8)),
        ],
        compiler_params=pltpu.CompilerParams(
            dimension_semantics=("arbitrary", "arbitrary"),
            vmem_limit_bytes=58 * 1024 * 1024,
        ),
        name="rwkv_wkv_scan",
    )(k2, v2, aa3, bb3, pp3, tf3, w3)


_OUT_TM = 256
_OUT_NI = (T // _OUT_TM) // 2


def _out_body(h_ref, r_ref, wkv_ref, wo_ref, o_ref):
    rw = r_ref[...] * wkv_ref[...]
    o_ref[...] = h_ref[...] + jnp.dot(rw, wo_ref[...],
                                      preferred_element_type=jnp.float32)


def _output(h, r, wkv, wo):
    rows = pl.BlockSpec((_OUT_TM, H), lambda c, i: (c * _OUT_NI + i, 0))
    return pl.pallas_call(
        _out_body,
        out_shape=jax.ShapeDtypeStruct((T, H), jnp.float32),
        grid=(2, _OUT_NI),
        in_specs=[rows, rows, rows,
                  pl.BlockSpec((H, H), lambda c, i: (0, 0))],
        out_specs=rows,
        compiler_params=pltpu.CompilerParams(
            dimension_semantics=("arbitrary", "arbitrary"),
            vmem_limit_bytes=58 * 1024 * 1024,
        ),
        name="rwkv_out",
    )(h, r, wkv, wo)


def kernel(hidden, sx, aa, bb, pp, time_decay, time_first, time_mix_key,
           time_mix_value, time_mix_receptance, Wk, Wv, Wr, Wo):
    s = jnp.float32(_LOG2E)
    tma = jnp.stack((time_mix_key * s, time_mix_value,
                     time_mix_receptance))
    tmb = jnp.stack(((1.0 - time_mix_key) * s, 1.0 - time_mix_value,
                     1.0 - time_mix_receptance))
    k, v, r = _projection(tma, tmb, sx[None, :], hidden, Wk, Wv, Wr)

    w_decay2 = -jnp.exp(time_decay) * _LOG2E
    tf2 = time_first * _LOG2E
    pp2 = pp * _LOG2E
    wkv, aa3, bb3, pp3 = _wkv_scan(
        k, v, aa.reshape(16, 128), bb.reshape(16, 128), pp2.reshape(16, 128),
        tf2.reshape(16, 128), w_decay2.reshape(16, 128))

    out = _output(hidden, r, wkv, Wo)
    return (out, hidden[-1, :], aa3.reshape(H), bb3.reshape(H),
            pp3.reshape(H) * jnp.float32(1.0 / _LOG2E))


# speedup vs baseline: 40.7364x; 1.0109x over previous
import math

import jax
import jax.numpy as jnp
from jax.experimental import pallas as pl
from jax.experimental.pallas import tpu as pltpu

T = 4096
H = 2048

_LOG2E = math.log2(math.e)


_PROJ_TM = 256
_PROJ_NI = T // _PROJ_TM


def _proj_body(tma_ref, tmb_ref, sx_ref, h_ref,
               wk_hbm, wv_hbm, wr_hbm,
               k_ref, v_ref, r_ref, wk_s, wv_s, wr_s, last_ref, sems):
    c = pl.program_id(0)
    i = pl.program_id(1)

    @pl.when(i == 0)
    def _():
        cols = pl.ds(c * (H // 2), H // 2)
        for n, (hbm, scr) in enumerate(((wk_hbm, wk_s), (wv_hbm, wv_s),
                                        (wr_hbm, wr_s))):
            pltpu.make_async_copy(hbm.at[:, cols], scr, sems.at[n]).start()
        for n, (hbm, scr) in enumerate(((wk_hbm, wk_s), (wv_hbm, wv_s),
                                        (wr_hbm, wr_s))):
            pltpu.make_async_copy(hbm.at[:, cols], scr, sems.at[n]).wait()

    h = h_ref[...]
    first = jnp.where(i == 0, sx_ref[...], last_ref[...])
    last_ref[...] = h[_PROJ_TM - 1:_PROJ_TM, :]
    cx = jnp.concatenate((first, h[:-1, :]), axis=0)
    xk = h * tma_ref[0:1] + cx * tmb_ref[0:1]
    xv = h * tma_ref[1:2] + cx * tmb_ref[1:2]
    xr = h * tma_ref[2:3] + cx * tmb_ref[2:3]
    k_ref[...] = jnp.dot(xk, wk_s[...], preferred_element_type=jnp.float32)
    v_ref[...] = jnp.dot(xv, wv_s[...], preferred_element_type=jnp.float32)
    r_ref[...] = jax.nn.sigmoid(
        jnp.dot(xr, wr_s[...], preferred_element_type=jnp.float32)
    ).astype(jnp.bfloat16)


def _projection(tma, tmb, sx2, h, wk, wv, wr):
    rows = pl.BlockSpec((_PROJ_TM, H), lambda c, i: (i, 0))
    wany = pl.BlockSpec(memory_space=pl.ANY)
    oblk = pl.BlockSpec((_PROJ_TM, H // 2), lambda c, i: (i, c))
    vec = pl.BlockSpec((3, H), lambda c, i: (0, 0))
    svec = pl.BlockSpec((1, H), lambda c, i: (0, 0))
    return pl.pallas_call(
        _proj_body,
        out_shape=(
            jax.ShapeDtypeStruct((T, H), jnp.float32),
            jax.ShapeDtypeStruct((T, H), jnp.float32),
            jax.ShapeDtypeStruct((T, H), jnp.bfloat16),
        ),
        grid=(2, _PROJ_NI),
        in_specs=[vec, vec, svec, rows, wany, wany, wany],
        out_specs=(oblk, oblk, oblk),
        scratch_shapes=[
            pltpu.VMEM((H, H // 2), jnp.float32),
            pltpu.VMEM((H, H // 2), jnp.float32),
            pltpu.VMEM((H, H // 2), jnp.float32),
            pltpu.VMEM((1, H), jnp.float32),
            pltpu.SemaphoreType.DMA((3,)),
        ],
        compiler_params=pltpu.CompilerParams(
            dimension_semantics=("arbitrary", "arbitrary"),
            vmem_limit_bytes=58 * 1024 * 1024,
        ),
        name="rwkv_proj",
    )(tma, tmb, sx2, h, wk, wv, wr)


_SC_TC = 512
_SC_NT = T // _SC_TC


def _in_copies(src_hbm, buf, sems, slot, chunk, c):
    t0 = chunk * _SC_TC
    for g in range(8):
        col = (c * 8 + g) * 128
        yield pltpu.make_async_copy(
            src_hbm.at[pl.ds(t0, _SC_TC), pl.ds(col, 128)],
            buf.at[slot, :, g, :],
            sems.at[slot, g])


def _out_copies(dst_hbm, buf, sems, slot, chunk, c):
    t0 = chunk * _SC_TC
    for g in range(8):
        col = (c * 8 + g) * 128
        yield pltpu.make_async_copy(
            buf.at[slot, :, g, :],
            dst_hbm.at[pl.ds(t0, _SC_TC), pl.ds(col, 128)],
            sems.at[slot, g])


def _scan_body(k_hbm, v_hbm, aa0_ref, bb0_ref, pp0_ref, tf_ref, w_ref,
               wkv_hbm, aa_ref, bb_ref, pp_ref,
               kbuf, vbuf, obuf, ksem, vsem, osem):
    c = pl.program_id(0)
    t = pl.program_id(1)
    slot = jax.lax.rem(t, 2)
    nslot = jax.lax.rem(t + 1, 2)

    @pl.when(t == 0)
    def _():
        for cp in _in_copies(k_hbm, kbuf, ksem, 0, 0, c):
            cp.start()
        for cp in _in_copies(v_hbm, vbuf, vsem, 0, 0, c):
            cp.start()
        aa_ref[...] = aa0_ref[...]
        bb_ref[...] = bb0_ref[...]
        pp_ref[...] = pp0_ref[...]

    @pl.when(t + 1 < _SC_NT)
    def _():
        for cp in _in_copies(k_hbm, kbuf, ksem, nslot, t + 1, c):
            cp.start()
        for cp in _in_copies(v_hbm, vbuf, vsem, nslot, t + 1, c):
            cp.start()

    @pl.when(t >= 2)
    def _():
        for cp in _out_copies(wkv_hbm, obuf, osem, slot, t - 2, c):
            cp.wait()

    for cp in _in_copies(k_hbm, kbuf, ksem, slot, t, c):
        cp.wait()
    for cp in _in_copies(v_hbm, vbuf, vsem, slot, t, c):
        cp.wait()

    tf = tf_ref[...]
    w = w_ref[...]

    def step(tt, carry):
        aa, bb, ipp = carry
        kk = kbuf[slot, tt]
        vv = vbuf[slot, tt]
        ww = tf + kk
        p = jnp.maximum(ipp, ww)
        e1 = jnp.exp2(ipp - p)
        e2 = jnp.exp2(ww - p)
        obuf[slot, tt] = (e1 * aa + e2 * vv) / (e1 * bb + e2)
        ww2 = w + ipp
        p2 = jnp.maximum(ww2, kk)
        e1b = jnp.exp2(ww2 - p2)
        e2b = jnp.exp2(kk - p2)
        return (e1b * aa + e2b * vv, e1b * bb + e2b, p2)

    init = (aa_ref[...], bb_ref[...], pp_ref[...])
    aa, bb, pp = jax.lax.fori_loop(0, _SC_TC, step, init, unroll=8)
    aa_ref[...] = aa
    bb_ref[...] = bb
    pp_ref[...] = pp

    for cp in _out_copies(wkv_hbm, obuf, osem, slot, t, c):
        cp.start()

    @pl.when(t == _SC_NT - 1)
    def _():
        for cp in _out_copies(wkv_hbm, obuf, osem, nslot, t - 1, c):
            cp.wait()
        for cp in _out_copies(wkv_hbm, obuf, osem, slot, t, c):
            cp.wait()


def _wkv_scan(k2, v2, aa3, bb3, pp3, tf3, w3):
    st = pl.BlockSpec((8, 128), lambda c, t: (c, 0))
    hbm = pl.BlockSpec(memory_space=pl.ANY)
    return pl.pallas_call(
        _scan_body,
        out_shape=(
            jax.ShapeDtypeStruct((T, H), jnp.float32),
            jax.ShapeDtypeStruct((16, 128), jnp.float32),
            jax.ShapeDtypeStruct((16, 128), jnp.float32),
            jax.ShapeDtypeStruct((16, 128), jnp.float32),
        ),
        grid=(2, _SC_NT),
        in_specs=[hbm, hbm, st, st, st, st, st],
        out_specs=(hbm, st, st, st),
        scratch_shapes=[
            pltpu.VMEM((2, _SC_TC, 8, 128), jnp.float32),
            pltpu.VMEM((2, _SC_TC, 8, 128), jnp.float32),
            pltpu.VMEM((2, _SC_TC, 8, 128), jnp.float32),
            pltpu.SemaphoreType.DMA((2, 8)),
            pltpu.SemaphoreType.DMA((2, 8)),
            pltpu.SemaphoreType.DMA((2, 8)),
        ],
        compiler_params=pltpu.CompilerParams(
            dimension_semantics=("arbitrary", "arbitrary"),
            vmem_limit_bytes=58 * 1024 * 1024,
        ),
        name="rwkv_wkv_scan",
    )(k2, v2, aa3, bb3, pp3, tf3, w3)


_OUT_TM = 512
_OUT_NI = (T // _OUT_TM) // 2


def _out_body(h_ref, r_ref, wkv_ref, wo_hbm, o_ref, wo_s, sem):
    c = pl.program_id(0)
    i = pl.program_id(1)

    @pl.when((c == 0) & (i == 0))
    def _():
        pltpu.make_async_copy(wo_hbm, wo_s, sem).start()
        pltpu.make_async_copy(wo_hbm, wo_s, sem).wait()

    rw = r_ref[...] * wkv_ref[...]
    o_ref[...] = h_ref[...] + jnp.dot(rw, wo_s[...],
                                      preferred_element_type=jnp.float32)


def _output(h, r, wkv, wo):
    rows = pl.BlockSpec((_OUT_TM, H), lambda c, i: (c * _OUT_NI + i, 0))
    return pl.pallas_call(
        _out_body,
        out_shape=jax.ShapeDtypeStruct((T, H), jnp.float32),
        grid=(2, _OUT_NI),
        in_specs=[rows, rows, rows, pl.BlockSpec(memory_space=pl.ANY)],
        out_specs=rows,
        scratch_shapes=[
            pltpu.VMEM((H, H), jnp.float32),
            pltpu.SemaphoreType.DMA,
        ],
        compiler_params=pltpu.CompilerParams(
            dimension_semantics=("arbitrary", "arbitrary"),
            vmem_limit_bytes=58 * 1024 * 1024,
        ),
        name="rwkv_out",
    )(h, r, wkv, wo)


def kernel(hidden, sx, aa, bb, pp, time_decay, time_first, time_mix_key,
           time_mix_value, time_mix_receptance, Wk, Wv, Wr, Wo):
    s = jnp.float32(_LOG2E)
    tma = jnp.stack((time_mix_key * s, time_mix_value,
                     time_mix_receptance))
    tmb = jnp.stack(((1.0 - time_mix_key) * s, 1.0 - time_mix_value,
                     1.0 - time_mix_receptance))
    k, v, r = _projection(tma, tmb, sx[None, :], hidden, Wk, Wv, Wr)

    w_decay2 = -jnp.exp(time_decay) * _LOG2E
    tf2 = time_first * _LOG2E
    pp2 = pp * _LOG2E
    wkv, aa3, bb3, pp3 = _wkv_scan(
        k, v, aa.reshape(16, 128), bb.reshape(16, 128), pp2.reshape(16, 128),
        tf2.reshape(16, 128), w_decay2.reshape(16, 128))

    out = _output(hidden, r, wkv, Wo)
    return (out, hidden[-1, :], aa3.reshape(H), bb3.reshape(H),
            pp3.reshape(H) * jnp.float32(1.0 / _LOG2E))
